# baseline (speedup 1.0000x reference)
import functools

import jax
import jax.numpy as jnp
from jax import lax
from jax.experimental import pallas as pl
from jax.experimental.pallas import tpu as pltpu
from jax.experimental.pallas import tpu_sc as plsc

ROUTED_SCALING = 2.5
BT = 128
NC, NS = 2, 16
NW = NC * NS
MAXB_PAD = 64


def _gate_body(x_ref, gw_ref, gb_ref, idx_ref, w_ref, bases_ref, meta_ref,
               cnt_ref):
    b = pl.program_id(0)
    nsteps = pl.num_programs(0)
    x = x_ref[...]
    B = x.shape[0]
    E = gw_ref.shape[0]

    @pl.when(b == 0)
    def _():
        cnt_ref[...] = jnp.zeros_like(cnt_ref)

    logits = lax.dot_general(x, gw_ref[...], (((1,), (1,)), ((), ())),
                             preferred_element_type=jnp.float32)
    scores = 1.0 / (1.0 + jnp.exp(-logits))
    sc = scores + gb_ref[...]
    l16 = lax.broadcasted_iota(jnp.int32, (B, E), 1)
    grp = l16 // 4
    NEGF = jnp.float32(-1e30)
    gcol = []
    for g in range(4):
        vg = jnp.where(grp == g, sc, NEGF)
        m1 = jnp.max(vg, axis=1, keepdims=True)
        am1 = jnp.min(jnp.where(vg == m1, l16, 99), axis=1, keepdims=True)
        m2 = jnp.max(jnp.where(l16 == am1, NEGF, vg), axis=1, keepdims=True)
        gcol.append(m1 + m2)
    M1 = jnp.maximum(jnp.maximum(gcol[0], gcol[1]),
                     jnp.maximum(gcol[2], gcol[3]))
    g1 = jnp.where(gcol[0] == M1, 0,
                   jnp.where(gcol[1] == M1, 1,
                             jnp.where(gcol[2] == M1, 2, 3)))
    mcol = [jnp.where(g1 == g, NEGF, gcol[g]) for g in range(4)]
    M2 = jnp.maximum(jnp.maximum(mcol[0], mcol[1]),
                     jnp.maximum(mcol[2], mcol[3]))
    g2 = jnp.where(mcol[0] == M2, 0,
                   jnp.where(mcol[1] == M2, 1,
                             jnp.where(mcol[2] == M2, 2, 3)))
    emask = (grp == g1) | (grp == g2)
    masked = jnp.where(emask, sc, 0.0)
    E1 = jnp.max(masked, axis=1, keepdims=True)
    e1 = jnp.min(jnp.where(masked == E1, l16, 99), axis=1, keepdims=True)
    masked2 = jnp.where(l16 == e1, NEGF, masked)
    E2 = jnp.max(masked2, axis=1, keepdims=True)
    e2 = jnp.min(jnp.where(masked2 == E2, l16, 99), axis=1, keepdims=True)
    w1v = jnp.sum(jnp.where(l16 == e1, scores, 0.0), axis=1, keepdims=True)
    w2v = jnp.sum(jnp.where(l16 == e2, scores, 0.0), axis=1, keepdims=True)
    den = w1v + w2v + 1e-20
    l2 = lax.broadcasted_iota(jnp.int32, (B, 2), 1)
    idx_ref[...] = jnp.where(l2 == 0, jnp.broadcast_to(e1, (B, 2)),
                             jnp.broadcast_to(e2, (B, 2)))
    w_ref[...] = jnp.where(l2 == 0,
                           jnp.broadcast_to(w1v / den, (B, 2)),
                           jnp.broadcast_to(w2v / den, (B, 2))) * ROUTED_SCALING
    oh = ((l16 == e1) | (l16 == e2)).astype(jnp.float32)
    cnt_ref[...] = cnt_ref[...] + jnp.sum(oh, axis=0, keepdims=True)

    @pl.when(b == nsteps - 1)
    def _():
        counts = cnt_ref[...]
        nb = jnp.floor((counts + (BT - 1)) * (1.0 / BT))
        r16 = lax.broadcasted_iota(jnp.int32, (E, E), 0)
        c16 = lax.broadcasted_iota(jnp.int32, (E, E), 1)
        upper = (r16 <= c16).astype(jnp.float32)
        incl = lax.dot_general(nb, upper, (((1,), (0,)), ((), ())),
                               preferred_element_type=jnp.float32)
        bstart = incl - nb
        bases_ref[...] = bstart * BT
        nvalid = jnp.sum(nb, axis=1, keepdims=True)
        eye = (r16 == c16).astype(jnp.float32)
        bstart_col = lax.dot_general(eye, bstart, (((1,), (1,)), ((), ())),
                                     preferred_element_type=jnp.float32)
        iota_b = lax.broadcasted_iota(
            jnp.int32, (E, MAXB_PAD), 1).astype(jnp.float32)
        ind = (iota_b >= bstart_col).astype(jnp.float32)
        ones = jnp.ones((1, E), jnp.float32)
        be = lax.dot_general(ones, ind, (((1,), (0,)), ((), ())),
                             preferred_element_type=jnp.float32) - 1.0
        belast = jnp.sum((bstart < nvalid).astype(jnp.float32),
                         axis=1, keepdims=True) - 1.0
        be = jnp.minimum(be, belast)
        l64 = lax.broadcasted_iota(jnp.int32, (1, MAXB_PAD), 1)
        meta_ref[...] = jnp.where(
            l64 == 48, jnp.broadcast_to(nvalid, (1, MAXB_PAD)),
            be).astype(jnp.int32)


def _gate_call(x, gate_w, gate_bias):
    T, D = x.shape
    E = gate_w.shape[0]
    B1 = 128
    return pl.pallas_call(
        _gate_body,
        grid=(T // B1,),
        in_specs=[
            pl.BlockSpec((B1, D), lambda b: (b, 0)),
            pl.BlockSpec((E, D), lambda b: (0, 0)),
            pl.BlockSpec((1, E), lambda b: (0, 0)),
        ],
        out_specs=[
            pl.BlockSpec((B1, 2), lambda b: (b, 0)),
            pl.BlockSpec((B1, 2), lambda b: (b, 0)),
            pl.BlockSpec((1, E), lambda b: (0, 0)),
            pl.BlockSpec((1, MAXB_PAD), lambda b: (0, 0)),
        ],
        out_shape=[
            jax.ShapeDtypeStruct((T, 2), jnp.int32),
            jax.ShapeDtypeStruct((T, 2), jnp.float32),
            jax.ShapeDtypeStruct((1, E), jnp.float32),
            jax.ShapeDtypeStruct((1, MAXB_PAD), jnp.int32),
        ],
        scratch_shapes=[pltpu.VMEM((1, E), jnp.float32)],
    )(x, gate_w, gate_bias)


def _slot_body(ti_ref, bases_ref, s0_ref, s1_ref, cnt_ref):
    b = pl.program_id(0)
    B = ti_ref.shape[0]
    E = bases_ref.shape[1]

    @pl.when(b == 0)
    def _():
        cnt_ref[...] = jnp.zeros_like(cnt_ref)

    le = lax.broadcasted_iota(jnp.int32, (B, E), 1)
    rr = lax.broadcasted_iota(jnp.int32, (B, B), 0)
    cc = lax.broadcasted_iota(jnp.int32, (B, B), 1)
    ltri = (cc < rr).astype(jnp.float32)
    bases = bases_ref[...]
    oh0 = (ti_ref[:, 0:1] == le).astype(jnp.float32)
    oh1 = (ti_ref[:, 1:2] == le).astype(jnp.float32)
    pref0 = lax.dot_general(ltri, oh0, (((1,), (0,)), ((), ())),
                            preferred_element_type=jnp.float32) + cnt_ref[...]
    s0 = jnp.sum((pref0 + bases) * oh0, axis=1, keepdims=True)
    cnt1 = cnt_ref[...] + jnp.sum(oh0, axis=0, keepdims=True)
    pref1 = lax.dot_general(ltri, oh1, (((1,), (0,)), ((), ())),
                            preferred_element_type=jnp.float32) + cnt1
    s1 = jnp.sum((pref1 + bases) * oh1, axis=1, keepdims=True)
    cnt_ref[...] = cnt1 + jnp.sum(oh1, axis=0, keepdims=True)
    s0_ref[...] = s0.astype(jnp.int32)
    s1_ref[...] = s1.astype(jnp.int32)


def _slot_call(ti, bases):
    T = ti.shape[0]
    E = bases.shape[1]
    B1 = 128
    return pl.pallas_call(
        _slot_body,
        grid=(T // B1,),
        in_specs=[
            pl.BlockSpec((B1, 2), lambda b: (b, 0)),
            pl.BlockSpec((1, E), lambda b: (0, 0)),
        ],
        out_specs=[
            pl.BlockSpec((B1, 1), lambda b: (b, 0)),
            pl.BlockSpec((B1, 1), lambda b: (b, 0)),
        ],
        out_shape=[
            jax.ShapeDtypeStruct((T, 1), jnp.int32),
            jax.ShapeDtypeStruct((T, 1), jnp.int32),
        ],
        scratch_shapes=[pltpu.VMEM((1, E), jnp.float32)],
    )(ti, bases)


def _scatter_rows(x, idx0, idx1, S):
    T, D = x.shape
    t_per_w = T // NW
    mesh = plsc.VectorSubcoreMesh(core_axis_name="c", subcore_axis_name="s")

    @functools.partial(
        pl.kernel, mesh=mesh,
        out_type=jax.ShapeDtypeStruct((S, D), jnp.float32),
        scratch_types=[
            pltpu.VMEM((t_per_w,), jnp.int32),
            pltpu.VMEM((t_per_w,), jnp.int32),
            pltpu.VMEM((t_per_w, D), jnp.float32),
            pltpu.SemaphoreType.DMA,
        ],
    )
    def k(x_hbm, i0_hbm, i1_hbm, out_hbm, i0_v, i1_v, rows_v, sem):
        wid = lax.axis_index("s") * NC + lax.axis_index("c")
        base = wid * t_per_w
        pltpu.sync_copy(x_hbm.at[pl.ds(base, t_per_w)], rows_v)
        pltpu.sync_copy(i0_hbm.at[wid], i0_v)
        pltpu.sync_copy(i1_hbm.at[wid], i1_v)
        pltpu.async_copy(rows_v, out_hbm.at[i0_v], sem)
        pltpu.async_copy(rows_v, out_hbm.at[i1_v], sem)
        pltpu.make_async_copy(rows_v, out_hbm.at[i0_v], sem).wait()
        pltpu.make_async_copy(rows_v, out_hbm.at[i1_v], sem).wait()

    return k(x, idx0.reshape(NW, t_per_w), idx1.reshape(NW, t_per_w))


def _gather_rows(table, idx, chunk):
    B = idx.shape[0]
    D = table.shape[1]
    b_per_w = B // NW
    nch = b_per_w // chunk
    mesh = plsc.VectorSubcoreMesh(core_axis_name="c", subcore_axis_name="s")

    @functools.partial(
        pl.kernel, mesh=mesh,
        out_type=jax.ShapeDtypeStruct((B, D), jnp.float32),
        scratch_types=[
            pltpu.VMEM((chunk,), jnp.int32),
            pltpu.VMEM((chunk, D), jnp.float32),
            pltpu.SemaphoreType.DMA,
        ],
    )
    def k(table_hbm, idx_hbm, out_hbm, idx_v, rows_v, sem):
        wid = lax.axis_index("s") * NC + lax.axis_index("c")
        base = wid * b_per_w
        for i in range(nch):
            off = base + i * chunk
            pltpu.sync_copy(idx_hbm.at[pl.ds(off, chunk)], idx_v)
            pltpu.async_copy(table_hbm.at[idx_v], rows_v, sem).wait()
            pltpu.sync_copy(rows_v, out_hbm.at[pl.ds(off, chunk)])

    return k(table, idx)


def _expert_body(meta_ref, xs_ref, w1_ref, w2_ref, ys_ref):
    b = pl.program_id(0)
    nvalid = meta_ref[0, 48]

    @pl.when(b < nvalid)
    def _():
        h = lax.dot_general(xs_ref[...], w1_ref[0], (((1,), (1,)), ((), ())),
                            preferred_element_type=jnp.float32)
        h = jnp.maximum(h, 0.0)
        h = h * h
        ys_ref[...] = lax.dot_general(h, w2_ref[0], (((1,), (1,)), ((), ())),
                                      preferred_element_type=jnp.float32)


def _expert_call(meta, xs, w1, w2, maxb):
    S, D = xs.shape
    E, I, _ = w1.shape
    grid_spec = pltpu.PrefetchScalarGridSpec(
        num_scalar_prefetch=1,
        grid=(maxb,),
        in_specs=[
            pl.BlockSpec((BT, D), lambda b, m: (b, 0)),
            pl.BlockSpec((1, I, D), lambda b, m: (m[0, b], 0, 0)),
            pl.BlockSpec((1, D, I), lambda b, m: (m[0, b], 0, 0)),
        ],
        out_specs=pl.BlockSpec((BT, D), lambda b, m: (b, 0)),
    )
    return pl.pallas_call(
        _expert_body,
        grid_spec=grid_spec,
        out_shape=jax.ShapeDtypeStruct((S, D), jnp.float32),
    )(meta, xs, w1, w2)


def _combine_body(x_ref, sw1_ref, sw2_ref, y0_ref, y1_ref, tw_ref, o_ref):
    h = lax.dot_general(x_ref[...], sw1_ref[...], (((1,), (1,)), ((), ())),
                        preferred_element_type=jnp.float32)
    h = jnp.maximum(h, 0.0)
    h = h * h
    sh = lax.dot_general(h, sw2_ref[...], (((1,), (1,)), ((), ())),
                         preferred_element_type=jnp.float32)
    w = tw_ref[...]
    o_ref[...] = sh + w[:, 0:1] * y0_ref[...] + w[:, 1:2] * y1_ref[...]


def _combine_call(x, shared_w1, shared_w2, yg, tw):
    T, D = x.shape
    SI = shared_w1.shape[0]
    nb = T // BT
    return pl.pallas_call(
        _combine_body,
        grid=(nb,),
        in_specs=[
            pl.BlockSpec((BT, D), lambda b: (b, 0)),
            pl.BlockSpec((SI, D), lambda b: (0, 0)),
            pl.BlockSpec((D, SI), lambda b: (0, 0)),
            pl.BlockSpec((BT, D), lambda b: (b, 0)),
            pl.BlockSpec((BT, D), lambda b: (b + nb, 0)),
            pl.BlockSpec((BT, 2), lambda b: (b, 0)),
        ],
        out_specs=pl.BlockSpec((BT, D), lambda b: (b, 0)),
        out_shape=jax.ShapeDtypeStruct((T, D), jnp.float32),
    )(x, shared_w1, shared_w2, yg, yg, tw)


def kernel(hidden_states, gate_w, gate_bias, w1, w2, shared_w1, shared_w2):
    x = hidden_states
    T, D = x.shape
    E = gate_w.shape[0]
    P = 2 * T
    maxb = P // BT + E
    S = maxb * BT

    ti, tw, bases, meta = _gate_call(x, gate_w, gate_bias.reshape(1, E))
    slot0, slot1 = _slot_call(ti, bases)

    xs = _scatter_rows(x, slot0, slot1, S)
    ys = _expert_call(meta, xs, w1, w2, maxb)
    idx_comb = jnp.concatenate([slot0, slot1], axis=0).reshape(P)
    yg = _gather_rows(ys, idx_comb, 64)

    return _combine_call(x, shared_w1, shared_w2, yg, tw)


# speedup vs baseline: 1.1190x; 1.1190x over previous
import functools

import jax
import jax.numpy as jnp
from jax import lax
from jax.experimental import pallas as pl
from jax.experimental.pallas import tpu as pltpu
from jax.experimental.pallas import tpu_sc as plsc

ROUTED_SCALING = 2.5
BT = 128
NC, NS = 2, 16
NW = NC * NS
MAXB_PAD = 64


def _gate_body(x_ref, gw_ref, gb_ref, idx_ref, w_ref, bases_ref, meta_ref,
               cnt_ref):
    b = pl.program_id(0)
    nsteps = pl.num_programs(0)
    x = x_ref[...]
    B = x.shape[0]
    E = gw_ref.shape[0]

    @pl.when(b == 0)
    def _():
        cnt_ref[...] = jnp.zeros_like(cnt_ref)

    logits = lax.dot_general(x, gw_ref[...], (((1,), (1,)), ((), ())),
                             preferred_element_type=jnp.float32)
    scores = 1.0 / (1.0 + jnp.exp(-logits))
    sc = scores + gb_ref[...]
    l16 = lax.broadcasted_iota(jnp.int32, (B, E), 1)
    grp = l16 // 4
    NEGF = jnp.float32(-1e30)
    gcol = []
    for g in range(4):
        vg = jnp.where(grp == g, sc, NEGF)
        m1 = jnp.max(vg, axis=1, keepdims=True)
        am1 = jnp.min(jnp.where(vg == m1, l16, 99), axis=1, keepdims=True)
        m2 = jnp.max(jnp.where(l16 == am1, NEGF, vg), axis=1, keepdims=True)
        gcol.append(m1 + m2)
    M1 = jnp.maximum(jnp.maximum(gcol[0], gcol[1]),
                     jnp.maximum(gcol[2], gcol[3]))
    g1 = jnp.where(gcol[0] == M1, 0,
                   jnp.where(gcol[1] == M1, 1,
                             jnp.where(gcol[2] == M1, 2, 3)))
    mcol = [jnp.where(g1 == g, NEGF, gcol[g]) for g in range(4)]
    M2 = jnp.maximum(jnp.maximum(mcol[0], mcol[1]),
                     jnp.maximum(mcol[2], mcol[3]))
    g2 = jnp.where(mcol[0] == M2, 0,
                   jnp.where(mcol[1] == M2, 1,
                             jnp.where(mcol[2] == M2, 2, 3)))
    emask = (grp == g1) | (grp == g2)
    masked = jnp.where(emask, sc, 0.0)
    E1 = jnp.max(masked, axis=1, keepdims=True)
    e1 = jnp.min(jnp.where(masked == E1, l16, 99), axis=1, keepdims=True)
    masked2 = jnp.where(l16 == e1, NEGF, masked)
    E2 = jnp.max(masked2, axis=1, keepdims=True)
    e2 = jnp.min(jnp.where(masked2 == E2, l16, 99), axis=1, keepdims=True)
    w1v = jnp.sum(jnp.where(l16 == e1, scores, 0.0), axis=1, keepdims=True)
    w2v = jnp.sum(jnp.where(l16 == e2, scores, 0.0), axis=1, keepdims=True)
    den = w1v + w2v + 1e-20
    l2 = lax.broadcasted_iota(jnp.int32, (B, 2), 1)
    idx_ref[...] = jnp.where(l2 == 0, jnp.broadcast_to(e1, (B, 2)),
                             jnp.broadcast_to(e2, (B, 2)))
    w_ref[...] = jnp.where(l2 == 0,
                           jnp.broadcast_to(w1v / den, (B, 2)),
                           jnp.broadcast_to(w2v / den, (B, 2))) * ROUTED_SCALING
    oh = ((l16 == e1) | (l16 == e2)).astype(jnp.float32)
    cnt_ref[...] = cnt_ref[...] + jnp.sum(oh, axis=0, keepdims=True)

    @pl.when(b == nsteps - 1)
    def _():
        counts = cnt_ref[...]
        nb = jnp.floor((counts + (BT - 1)) * (1.0 / BT))
        r16 = lax.broadcasted_iota(jnp.int32, (E, E), 0)
        c16 = lax.broadcasted_iota(jnp.int32, (E, E), 1)
        upper = (r16 <= c16).astype(jnp.float32)
        incl = lax.dot_general(nb, upper, (((1,), (0,)), ((), ())),
                               preferred_element_type=jnp.float32)
        bstart = incl - nb
        bases_ref[...] = bstart * BT
        nvalid = jnp.sum(nb, axis=1, keepdims=True)
        eye = (r16 == c16).astype(jnp.float32)
        bstart_col = lax.dot_general(eye, bstart, (((1,), (1,)), ((), ())),
                                     preferred_element_type=jnp.float32)
        iota_b = lax.broadcasted_iota(
            jnp.int32, (E, MAXB_PAD), 1).astype(jnp.float32)
        ind = (iota_b >= bstart_col).astype(jnp.float32)
        ones = jnp.ones((1, E), jnp.float32)
        be = lax.dot_general(ones, ind, (((1,), (0,)), ((), ())),
                             preferred_element_type=jnp.float32) - 1.0
        belast = jnp.sum((bstart < nvalid).astype(jnp.float32),
                         axis=1, keepdims=True) - 1.0
        be = jnp.minimum(be, belast)
        l64 = lax.broadcasted_iota(jnp.int32, (1, MAXB_PAD), 1)
        meta_ref[...] = jnp.where(
            l64 == 48, jnp.broadcast_to(nvalid, (1, MAXB_PAD)),
            be).astype(jnp.int32)


def _gate_call(x, gate_w, gate_bias):
    T, D = x.shape
    E = gate_w.shape[0]
    B1 = 256
    return pl.pallas_call(
        _gate_body,
        grid=(T // B1,),
        in_specs=[
            pl.BlockSpec((B1, D), lambda b: (b, 0)),
            pl.BlockSpec((E, D), lambda b: (0, 0)),
            pl.BlockSpec((1, E), lambda b: (0, 0)),
        ],
        out_specs=[
            pl.BlockSpec((B1, 2), lambda b: (b, 0)),
            pl.BlockSpec((B1, 2), lambda b: (b, 0)),
            pl.BlockSpec((1, E), lambda b: (0, 0)),
            pl.BlockSpec((1, MAXB_PAD), lambda b: (0, 0)),
        ],
        out_shape=[
            jax.ShapeDtypeStruct((T, 2), jnp.int32),
            jax.ShapeDtypeStruct((T, 2), jnp.float32),
            jax.ShapeDtypeStruct((1, E), jnp.float32),
            jax.ShapeDtypeStruct((1, MAXB_PAD), jnp.int32),
        ],
        scratch_shapes=[pltpu.VMEM((1, E), jnp.float32)],
    )(x, gate_w, gate_bias)


def _slot_body(ti_ref, bases_ref, s0_ref, s1_ref, cnt_ref):
    b = pl.program_id(0)
    B = ti_ref.shape[0]
    E = bases_ref.shape[1]

    @pl.when(b == 0)
    def _():
        cnt_ref[...] = jnp.zeros_like(cnt_ref)

    le = lax.broadcasted_iota(jnp.int32, (B, E), 1)
    rr = lax.broadcasted_iota(jnp.int32, (B, B), 0)
    cc = lax.broadcasted_iota(jnp.int32, (B, B), 1)
    ltri = (cc < rr).astype(jnp.float32)
    bases = bases_ref[...]
    oh0 = (ti_ref[:, 0:1] == le).astype(jnp.float32)
    oh1 = (ti_ref[:, 1:2] == le).astype(jnp.float32)
    pref0 = lax.dot_general(ltri, oh0, (((1,), (0,)), ((), ())),
                            preferred_element_type=jnp.float32) + cnt_ref[...]
    s0 = jnp.sum((pref0 + bases) * oh0, axis=1, keepdims=True)
    cnt1 = cnt_ref[...] + jnp.sum(oh0, axis=0, keepdims=True)
    pref1 = lax.dot_general(ltri, oh1, (((1,), (0,)), ((), ())),
                            preferred_element_type=jnp.float32) + cnt1
    s1 = jnp.sum((pref1 + bases) * oh1, axis=1, keepdims=True)
    cnt_ref[...] = cnt1 + jnp.sum(oh1, axis=0, keepdims=True)
    s0_ref[...] = s0.astype(jnp.int32)
    s1_ref[...] = s1.astype(jnp.int32)


def _slot_call(ti, bases):
    T = ti.shape[0]
    E = bases.shape[1]
    B1 = 256
    return pl.pallas_call(
        _slot_body,
        grid=(T // B1,),
        in_specs=[
            pl.BlockSpec((B1, 2), lambda b: (b, 0)),
            pl.BlockSpec((1, E), lambda b: (0, 0)),
        ],
        out_specs=[
            pl.BlockSpec((B1, 1), lambda b: (b, 0)),
            pl.BlockSpec((B1, 1), lambda b: (b, 0)),
        ],
        out_shape=[
            jax.ShapeDtypeStruct((T, 1), jnp.int32),
            jax.ShapeDtypeStruct((T, 1), jnp.int32),
        ],
        scratch_shapes=[pltpu.VMEM((1, E), jnp.float32)],
    )(ti, bases)


def _scatter_rows(x, idx0, idx1, S):
    T, D = x.shape
    t_per_w = T // NW
    mesh = plsc.VectorSubcoreMesh(core_axis_name="c", subcore_axis_name="s")

    @functools.partial(
        pl.kernel, mesh=mesh,
        out_type=jax.ShapeDtypeStruct((S, D), jnp.float32),
        scratch_types=[
            pltpu.VMEM((t_per_w,), jnp.int32),
            pltpu.VMEM((t_per_w,), jnp.int32),
            pltpu.VMEM((t_per_w, D), jnp.float32),
            pltpu.SemaphoreType.DMA,
        ],
    )
    def k(x_hbm, i0_hbm, i1_hbm, out_hbm, i0_v, i1_v, rows_v, sem):
        wid = lax.axis_index("s") * NC + lax.axis_index("c")
        base = wid * t_per_w
        pltpu.sync_copy(x_hbm.at[pl.ds(base, t_per_w)], rows_v)
        pltpu.sync_copy(i0_hbm.at[wid], i0_v)
        pltpu.sync_copy(i1_hbm.at[wid], i1_v)
        pltpu.async_copy(rows_v, out_hbm.at[i0_v], sem)
        pltpu.async_copy(rows_v, out_hbm.at[i1_v], sem)
        pltpu.make_async_copy(rows_v, out_hbm.at[i0_v], sem).wait()
        pltpu.make_async_copy(rows_v, out_hbm.at[i1_v], sem).wait()

    return k(x, idx0.reshape(NW, t_per_w), idx1.reshape(NW, t_per_w))


def _gather_rows(table, idx, chunk):
    B = idx.shape[0]
    D = table.shape[1]
    b_per_w = B // NW
    nch = b_per_w // chunk
    mesh = plsc.VectorSubcoreMesh(core_axis_name="c", subcore_axis_name="s")

    @functools.partial(
        pl.kernel, mesh=mesh,
        out_type=jax.ShapeDtypeStruct((B, D), jnp.float32),
        scratch_types=[
            pltpu.VMEM((chunk,), jnp.int32),
            pltpu.VMEM((chunk, D), jnp.float32),
            pltpu.SemaphoreType.DMA,
        ],
    )
    def k(table_hbm, idx_hbm, out_hbm, idx_v, rows_v, sem):
        wid = lax.axis_index("s") * NC + lax.axis_index("c")
        base = wid * b_per_w
        for i in range(nch):
            off = base + i * chunk
            pltpu.sync_copy(idx_hbm.at[pl.ds(off, chunk)], idx_v)
            pltpu.async_copy(table_hbm.at[idx_v], rows_v, sem).wait()
            pltpu.sync_copy(rows_v, out_hbm.at[pl.ds(off, chunk)])

    return k(table, idx)


def _expert_body(meta_ref, xs_ref, w1_ref, w2_ref, ys_ref):
    b = pl.program_id(0)
    nvalid = meta_ref[0, 48]

    @pl.when(b < nvalid)
    def _():
        h = lax.dot_general(xs_ref[...], w1_ref[0], (((1,), (1,)), ((), ())),
                            preferred_element_type=jnp.float32)
        h = jnp.maximum(h, 0.0)
        h = h * h
        ys_ref[...] = lax.dot_general(h, w2_ref[0], (((1,), (1,)), ((), ())),
                                      preferred_element_type=jnp.float32)


def _expert_call(meta, xs, w1, w2, maxb):
    S, D = xs.shape
    E, I, _ = w1.shape
    grid_spec = pltpu.PrefetchScalarGridSpec(
        num_scalar_prefetch=1,
        grid=(maxb,),
        in_specs=[
            pl.BlockSpec((BT, D), lambda b, m: (b, 0)),
            pl.BlockSpec((1, I, D), lambda b, m: (m[0, b], 0, 0)),
            pl.BlockSpec((1, D, I), lambda b, m: (m[0, b], 0, 0)),
        ],
        out_specs=pl.BlockSpec((BT, D), lambda b, m: (b, 0)),
    )
    return pl.pallas_call(
        _expert_body,
        grid_spec=grid_spec,
        out_shape=jax.ShapeDtypeStruct((S, D), jnp.float32),
    )(meta, xs, w1, w2)


def _shared_body(x_ref, sw1_ref, sw2_ref, sh_ref):
    h = lax.dot_general(x_ref[...], sw1_ref[...], (((1,), (1,)), ((), ())),
                        preferred_element_type=jnp.float32)
    h = jnp.maximum(h, 0.0)
    h = h * h
    sh_ref[...] = lax.dot_general(h, sw2_ref[...], (((1,), (1,)), ((), ())),
                                  preferred_element_type=jnp.float32)


def _shared_call(x, shared_w1, shared_w2):
    T, D = x.shape
    SI = shared_w1.shape[0]
    B1 = 256
    return pl.pallas_call(
        _shared_body,
        grid=(T // B1,),
        in_specs=[
            pl.BlockSpec((B1, D), lambda b: (b, 0)),
            pl.BlockSpec((SI, D), lambda b: (0, 0)),
            pl.BlockSpec((D, SI), lambda b: (0, 0)),
        ],
        out_specs=pl.BlockSpec((B1, D), lambda b: (b, 0)),
        out_shape=jax.ShapeDtypeStruct((T, D), jnp.float32),
    )(x, shared_w1, shared_w2)


def _combine_body(sh_ref, y0_ref, y1_ref, tw_ref, o_ref):
    w = tw_ref[...]
    o_ref[...] = (sh_ref[...] + w[:, 0:1] * y0_ref[...]
                  + w[:, 1:2] * y1_ref[...])


def _combine_call(sh, yg, tw):
    T, D = sh.shape
    nb = T // BT
    return pl.pallas_call(
        _combine_body,
        grid=(nb,),
        in_specs=[
            pl.BlockSpec((BT, D), lambda b: (b, 0)),
            pl.BlockSpec((BT, D), lambda b: (b, 0)),
            pl.BlockSpec((BT, D), lambda b: (b + nb, 0)),
            pl.BlockSpec((BT, 2), lambda b: (b, 0)),
        ],
        out_specs=pl.BlockSpec((BT, D), lambda b: (b, 0)),
        out_shape=jax.ShapeDtypeStruct((T, D), jnp.float32),
    )(sh, yg, yg, tw)


def kernel(hidden_states, gate_w, gate_bias, w1, w2, shared_w1, shared_w2):
    x = hidden_states
    T, D = x.shape
    E = gate_w.shape[0]
    P = 2 * T
    maxb = P // BT + E
    S = maxb * BT

    ti, tw, bases, meta = _gate_call(x, gate_w, gate_bias.reshape(1, E))
    slot0, slot1 = _slot_call(ti, bases)

    xs = _scatter_rows(x, slot0, slot1, S)
    sh = _shared_call(x, shared_w1, shared_w2)
    ys = _expert_call(meta, xs, w1, w2, maxb)
    idx_comb = jnp.concatenate([slot0, slot1], axis=0).reshape(P)
    yg = _gather_rows(ys, idx_comb, 64)

    return _combine_call(sh, yg, tw)


# speedup vs baseline: 1.3465x; 1.2033x over previous
import functools

import jax
import jax.numpy as jnp
from jax import lax
from jax.experimental import pallas as pl
from jax.experimental.pallas import tpu as pltpu
from jax.experimental.pallas import tpu_sc as plsc

ROUTED_SCALING = 2.5
BT = 256
NC, NS = 2, 16
NW = NC * NS
MAXB_PAD = 64


def _gate_body(x_ref, gw_ref, gb_ref, idx_ref, w_ref, bases_ref, meta_ref,
               cnt_ref):
    b = pl.program_id(0)
    nsteps = pl.num_programs(0)
    x = x_ref[...]
    B = x.shape[0]
    E = gw_ref.shape[0]

    @pl.when(b == 0)
    def _():
        cnt_ref[...] = jnp.zeros_like(cnt_ref)

    logits = lax.dot_general(x, gw_ref[...], (((1,), (1,)), ((), ())),
                             preferred_element_type=jnp.float32)
    scores = 1.0 / (1.0 + jnp.exp(-logits))
    sc = scores + gb_ref[...]
    l16 = lax.broadcasted_iota(jnp.int32, (B, E), 1)
    grp = l16 // 4
    NEGF = jnp.float32(-1e30)
    gcol = []
    for g in range(4):
        vg = jnp.where(grp == g, sc, NEGF)
        m1 = jnp.max(vg, axis=1, keepdims=True)
        am1 = jnp.min(jnp.where(vg == m1, l16, 99), axis=1, keepdims=True)
        m2 = jnp.max(jnp.where(l16 == am1, NEGF, vg), axis=1, keepdims=True)
        gcol.append(m1 + m2)
    M1 = jnp.maximum(jnp.maximum(gcol[0], gcol[1]),
                     jnp.maximum(gcol[2], gcol[3]))
    g1 = jnp.where(gcol[0] == M1, 0,
                   jnp.where(gcol[1] == M1, 1,
                             jnp.where(gcol[2] == M1, 2, 3)))
    mcol = [jnp.where(g1 == g, NEGF, gcol[g]) for g in range(4)]
    M2 = jnp.maximum(jnp.maximum(mcol[0], mcol[1]),
                     jnp.maximum(mcol[2], mcol[3]))
    g2 = jnp.where(mcol[0] == M2, 0,
                   jnp.where(mcol[1] == M2, 1,
                             jnp.where(mcol[2] == M2, 2, 3)))
    emask = (grp == g1) | (grp == g2)
    masked = jnp.where(emask, sc, 0.0)
    E1 = jnp.max(masked, axis=1, keepdims=True)
    e1 = jnp.min(jnp.where(masked == E1, l16, 99), axis=1, keepdims=True)
    masked2 = jnp.where(l16 == e1, NEGF, masked)
    E2 = jnp.max(masked2, axis=1, keepdims=True)
    e2 = jnp.min(jnp.where(masked2 == E2, l16, 99), axis=1, keepdims=True)
    w1v = jnp.sum(jnp.where(l16 == e1, scores, 0.0), axis=1, keepdims=True)
    w2v = jnp.sum(jnp.where(l16 == e2, scores, 0.0), axis=1, keepdims=True)
    den = w1v + w2v + 1e-20
    l2 = lax.broadcasted_iota(jnp.int32, (B, 2), 1)
    idx_ref[...] = jnp.where(l2 == 0, jnp.broadcast_to(e1, (B, 2)),
                             jnp.broadcast_to(e2, (B, 2)))
    w_ref[...] = jnp.where(l2 == 0,
                           jnp.broadcast_to(w1v / den, (B, 2)),
                           jnp.broadcast_to(w2v / den, (B, 2))) * ROUTED_SCALING
    oh = ((l16 == e1) | (l16 == e2)).astype(jnp.float32)
    cnt_ref[...] = cnt_ref[...] + jnp.sum(oh, axis=0, keepdims=True)

    @pl.when(b == nsteps - 1)
    def _():
        counts = cnt_ref[...]
        nb = jnp.floor((counts + (BT - 1)) * (1.0 / BT))
        r16 = lax.broadcasted_iota(jnp.int32, (E, E), 0)
        c16 = lax.broadcasted_iota(jnp.int32, (E, E), 1)
        upper = (r16 <= c16).astype(jnp.float32)
        incl = lax.dot_general(nb, upper, (((1,), (0,)), ((), ())),
                               preferred_element_type=jnp.float32)
        bstart = incl - nb
        bases_ref[...] = bstart * BT
        nvalid = jnp.sum(nb, axis=1, keepdims=True)
        eye = (r16 == c16).astype(jnp.float32)
        bstart_col = lax.dot_general(eye, bstart, (((1,), (1,)), ((), ())),
                                     preferred_element_type=jnp.float32)
        iota_b = lax.broadcasted_iota(
            jnp.int32, (E, MAXB_PAD), 1).astype(jnp.float32)
        ind = (iota_b >= bstart_col).astype(jnp.float32)
        ones = jnp.ones((1, E), jnp.float32)
        be = lax.dot_general(ones, ind, (((1,), (0,)), ((), ())),
                             preferred_element_type=jnp.float32) - 1.0
        belast = jnp.sum((bstart < nvalid).astype(jnp.float32),
                         axis=1, keepdims=True) - 1.0
        be = jnp.minimum(be, belast)
        l64 = lax.broadcasted_iota(jnp.int32, (1, MAXB_PAD), 1)
        meta_ref[...] = jnp.where(
            l64 == 48, jnp.broadcast_to(nvalid, (1, MAXB_PAD)),
            be).astype(jnp.int32)


def _gate_call(x, gate_w, gate_bias):
    T, D = x.shape
    E = gate_w.shape[0]
    B1 = 256
    return pl.pallas_call(
        _gate_body,
        grid=(T // B1,),
        in_specs=[
            pl.BlockSpec((B1, D), lambda b: (b, 0)),
            pl.BlockSpec((E, D), lambda b: (0, 0)),
            pl.BlockSpec((1, E), lambda b: (0, 0)),
        ],
        out_specs=[
            pl.BlockSpec((B1, 2), lambda b: (b, 0)),
            pl.BlockSpec((B1, 2), lambda b: (b, 0)),
            pl.BlockSpec((1, E), lambda b: (0, 0)),
            pl.BlockSpec((1, MAXB_PAD), lambda b: (0, 0)),
        ],
        out_shape=[
            jax.ShapeDtypeStruct((T, 2), jnp.int32),
            jax.ShapeDtypeStruct((T, 2), jnp.float32),
            jax.ShapeDtypeStruct((1, E), jnp.float32),
            jax.ShapeDtypeStruct((1, MAXB_PAD), jnp.int32),
        ],
        scratch_shapes=[pltpu.VMEM((1, E), jnp.float32)],
    )(x, gate_w, gate_bias)


def _slot_body(ti_ref, bases_ref, s0_ref, cnt_ref):
    b = pl.program_id(0)
    B = ti_ref.shape[0]
    E = bases_ref.shape[1]

    @pl.when(b == 0)
    def _():
        cnt_ref[...] = jnp.zeros_like(cnt_ref)

    le = lax.broadcasted_iota(jnp.int32, (B, E), 1)
    rr = lax.broadcasted_iota(jnp.int32, (B, B), 0)
    cc = lax.broadcasted_iota(jnp.int32, (B, B), 1)
    ltri = (cc < rr).astype(jnp.float32)
    bases = bases_ref[...]
    oh0 = (ti_ref[:, 0:1] == le).astype(jnp.float32)
    oh1 = (ti_ref[:, 1:2] == le).astype(jnp.float32)
    pref0 = lax.dot_general(ltri, oh0, (((1,), (0,)), ((), ())),
                            preferred_element_type=jnp.float32) + cnt_ref[...]
    s0 = jnp.sum((pref0 + bases) * oh0, axis=1, keepdims=True)
    cnt1 = cnt_ref[...] + jnp.sum(oh0, axis=0, keepdims=True)
    pref1 = lax.dot_general(ltri, oh1, (((1,), (0,)), ((), ())),
                            preferred_element_type=jnp.float32) + cnt1
    s1 = jnp.sum((pref1 + bases) * oh1, axis=1, keepdims=True)
    cnt_ref[...] = cnt1 + jnp.sum(oh1, axis=0, keepdims=True)
    s0_ref[0] = s0.astype(jnp.int32)
    s0_ref[1] = s1.astype(jnp.int32)


def _slot_call(ti, bases):
    T = ti.shape[0]
    E = bases.shape[1]
    B1 = 256
    return pl.pallas_call(
        _slot_body,
        grid=(T // B1,),
        in_specs=[
            pl.BlockSpec((B1, 2), lambda b: (b, 0)),
            pl.BlockSpec((1, E), lambda b: (0, 0)),
        ],
        out_specs=pl.BlockSpec((2, B1, 1), lambda b: (0, b, 0)),
        out_shape=jax.ShapeDtypeStruct((2, T, 1), jnp.int32),
        scratch_shapes=[pltpu.VMEM((1, E), jnp.float32)],
    )(ti, bases)


def _scatter_rows(x, idx0, idx1, S):
    T, D = x.shape
    t_per_w = T // NW
    mesh = plsc.VectorSubcoreMesh(core_axis_name="c", subcore_axis_name="s")

    @functools.partial(
        pl.kernel, mesh=mesh,
        out_type=jax.ShapeDtypeStruct((S, D), jnp.float32),
        scratch_types=[
            pltpu.VMEM((t_per_w,), jnp.int32),
            pltpu.VMEM((t_per_w,), jnp.int32),
            pltpu.VMEM((t_per_w, D), jnp.float32),
            pltpu.SemaphoreType.DMA,
        ],
    )
    def k(x_hbm, i0_hbm, i1_hbm, out_hbm, i0_v, i1_v, rows_v, sem):
        wid = lax.axis_index("s") * NC + lax.axis_index("c")
        base = wid * t_per_w
        pltpu.sync_copy(x_hbm.at[pl.ds(base, t_per_w)], rows_v)
        pltpu.sync_copy(i0_hbm.at[wid], i0_v)
        pltpu.sync_copy(i1_hbm.at[wid], i1_v)
        pltpu.async_copy(rows_v, out_hbm.at[i0_v], sem)
        pltpu.async_copy(rows_v, out_hbm.at[i1_v], sem)
        pltpu.make_async_copy(rows_v, out_hbm.at[i0_v], sem).wait()
        pltpu.make_async_copy(rows_v, out_hbm.at[i1_v], sem).wait()

    return k(x, idx0.reshape(NW, t_per_w), idx1.reshape(NW, t_per_w))


def _gather_rows(table, idx, chunk):
    B = idx.shape[0]
    D = table.shape[1]
    b_per_w = B // NW
    nch = b_per_w // chunk
    mesh = plsc.VectorSubcoreMesh(core_axis_name="c", subcore_axis_name="s")

    @functools.partial(
        pl.kernel, mesh=mesh,
        out_type=jax.ShapeDtypeStruct((B, D), jnp.float32),
        scratch_types=[
            pltpu.VMEM((chunk,), jnp.int32),
            pltpu.VMEM((chunk, D), jnp.float32),
            pltpu.SemaphoreType.DMA,
        ],
    )
    def k(table_hbm, idx_hbm, out_hbm, idx_v, rows_v, sem):
        wid = lax.axis_index("s") * NC + lax.axis_index("c")
        base = wid * b_per_w
        for i in range(nch):
            off = base + i * chunk
            pltpu.sync_copy(idx_hbm.at[pl.ds(off, chunk)], idx_v)
            pltpu.async_copy(table_hbm.at[idx_v], rows_v, sem).wait()
            pltpu.sync_copy(rows_v, out_hbm.at[pl.ds(off, chunk)])

    return k(table, idx)


def _expert_body(meta_ref, xs_ref, w1_ref, w2_ref, ys_ref):
    b = pl.program_id(0)
    nvalid = meta_ref[0, 48]

    @pl.when(b < nvalid)
    def _():
        h = lax.dot_general(xs_ref[...], w1_ref[0], (((1,), (1,)), ((), ())),
                            preferred_element_type=jnp.float32)
        h = jnp.maximum(h, 0.0)
        h = h * h
        ys_ref[...] = lax.dot_general(h, w2_ref[0], (((1,), (1,)), ((), ())),
                                      preferred_element_type=jnp.float32)


def _expert_call(meta, xs, w1, w2, maxb):
    S, D = xs.shape
    E, I, _ = w1.shape
    grid_spec = pltpu.PrefetchScalarGridSpec(
        num_scalar_prefetch=1,
        grid=(maxb,),
        in_specs=[
            pl.BlockSpec((BT, D),
                         lambda b, m: (jnp.minimum(b, m[0, 48] - 1), 0)),
            pl.BlockSpec((1, I, D), lambda b, m: (m[0, b], 0, 0)),
            pl.BlockSpec((1, D, I), lambda b, m: (m[0, b], 0, 0)),
        ],
        out_specs=pl.BlockSpec((BT, D),
                               lambda b, m: (jnp.minimum(b, m[0, 48] - 1), 0)),
    )
    return pl.pallas_call(
        _expert_body,
        grid_spec=grid_spec,
        out_shape=jax.ShapeDtypeStruct((S, D), jnp.float32),
    )(meta, xs, w1, w2)


def _shared_body(x_ref, sw1_ref, sw2_ref, sh_ref):
    h = lax.dot_general(x_ref[...], sw1_ref[...], (((1,), (1,)), ((), ())),
                        preferred_element_type=jnp.float32)
    h = jnp.maximum(h, 0.0)
    h = h * h
    sh_ref[...] = lax.dot_general(h, sw2_ref[...], (((1,), (1,)), ((), ())),
                                  preferred_element_type=jnp.float32)


def _shared_call(x, shared_w1, shared_w2):
    T, D = x.shape
    SI = shared_w1.shape[0]
    B1 = 256
    return pl.pallas_call(
        _shared_body,
        grid=(T // B1,),
        in_specs=[
            pl.BlockSpec((B1, D), lambda b: (b, 0)),
            pl.BlockSpec((SI, D), lambda b: (0, 0)),
            pl.BlockSpec((D, SI), lambda b: (0, 0)),
        ],
        out_specs=pl.BlockSpec((B1, D), lambda b: (b, 0)),
        out_shape=jax.ShapeDtypeStruct((T, D), jnp.float32),
    )(x, shared_w1, shared_w2)


def _combine_body(sh_ref, y0_ref, y1_ref, tw_ref, o_ref):
    w = tw_ref[...]
    o_ref[...] = (sh_ref[...] + w[:, 0:1] * y0_ref[...]
                  + w[:, 1:2] * y1_ref[...])


def _combine_call(sh, yg, tw):
    T, D = sh.shape
    nb = T // BT
    return pl.pallas_call(
        _combine_body,
        grid=(nb,),
        in_specs=[
            pl.BlockSpec((BT, D), lambda b: (b, 0)),
            pl.BlockSpec((BT, D), lambda b: (b, 0)),
            pl.BlockSpec((BT, D), lambda b: (b + nb, 0)),
            pl.BlockSpec((BT, 2), lambda b: (b, 0)),
        ],
        out_specs=pl.BlockSpec((BT, D), lambda b: (b, 0)),
        out_shape=jax.ShapeDtypeStruct((T, D), jnp.float32),
    )(sh, yg, yg, tw)


def kernel(hidden_states, gate_w, gate_bias, w1, w2, shared_w1, shared_w2):
    x = hidden_states
    T, D = x.shape
    E = gate_w.shape[0]
    P = 2 * T
    maxb = P // BT + E
    S = maxb * BT

    ti, tw, bases, meta = _gate_call(x, gate_w, gate_bias.reshape(1, E))
    slots = _slot_call(ti, bases)

    xs = _scatter_rows(x, slots[0], slots[1], S)
    sh = _shared_call(x, shared_w1, shared_w2)
    ys = _expert_call(meta, xs, w1, w2, maxb)
    yg = _gather_rows(ys, slots.reshape(P), 64)

    return _combine_call(sh, yg, tw)


# speedup vs baseline: 1.3637x; 1.0128x over previous
import functools

import jax
import jax.numpy as jnp
from jax import lax
from jax.experimental import pallas as pl
from jax.experimental.pallas import tpu as pltpu
from jax.experimental.pallas import tpu_sc as plsc

ROUTED_SCALING = 2.5
BT = 256
NC, NS = 2, 16
NW = NC * NS
MAXB_PAD = 64


def _gate_body(x_ref, gw_ref, gb_ref, idx_ref, w_ref, bases_ref, meta_ref,
               cnt_ref):
    b = pl.program_id(0)
    nsteps = pl.num_programs(0)
    x = x_ref[...]
    B = x.shape[0]
    E = gw_ref.shape[0]

    @pl.when(b == 0)
    def _():
        cnt_ref[...] = jnp.zeros_like(cnt_ref)

    logits = lax.dot_general(x, gw_ref[...], (((1,), (1,)), ((), ())),
                             preferred_element_type=jnp.float32)
    scores = 1.0 / (1.0 + jnp.exp(-logits))
    sc = scores + gb_ref[...]
    l16 = lax.broadcasted_iota(jnp.int32, (B, E), 1)
    grp = l16 // 4
    NEGF = jnp.float32(-1e30)
    gcol = []
    for g in range(4):
        vg = jnp.where(grp == g, sc, NEGF)
        m1 = jnp.max(vg, axis=1, keepdims=True)
        am1 = jnp.min(jnp.where(vg == m1, l16, 99), axis=1, keepdims=True)
        m2 = jnp.max(jnp.where(l16 == am1, NEGF, vg), axis=1, keepdims=True)
        gcol.append(m1 + m2)
    M1 = jnp.maximum(jnp.maximum(gcol[0], gcol[1]),
                     jnp.maximum(gcol[2], gcol[3]))
    g1 = jnp.where(gcol[0] == M1, 0,
                   jnp.where(gcol[1] == M1, 1,
                             jnp.where(gcol[2] == M1, 2, 3)))
    mcol = [jnp.where(g1 == g, NEGF, gcol[g]) for g in range(4)]
    M2 = jnp.maximum(jnp.maximum(mcol[0], mcol[1]),
                     jnp.maximum(mcol[2], mcol[3]))
    g2 = jnp.where(mcol[0] == M2, 0,
                   jnp.where(mcol[1] == M2, 1,
                             jnp.where(mcol[2] == M2, 2, 3)))
    emask = (grp == g1) | (grp == g2)
    masked = jnp.where(emask, sc, 0.0)
    E1 = jnp.max(masked, axis=1, keepdims=True)
    e1 = jnp.min(jnp.where(masked == E1, l16, 99), axis=1, keepdims=True)
    masked2 = jnp.where(l16 == e1, NEGF, masked)
    E2 = jnp.max(masked2, axis=1, keepdims=True)
    e2 = jnp.min(jnp.where(masked2 == E2, l16, 99), axis=1, keepdims=True)
    w1v = jnp.sum(jnp.where(l16 == e1, scores, 0.0), axis=1, keepdims=True)
    w2v = jnp.sum(jnp.where(l16 == e2, scores, 0.0), axis=1, keepdims=True)
    den = w1v + w2v + 1e-20
    l2 = lax.broadcasted_iota(jnp.int32, (B, 2), 1)
    idx_ref[...] = jnp.where(l2 == 0, jnp.broadcast_to(e1, (B, 2)),
                             jnp.broadcast_to(e2, (B, 2)))
    w_ref[...] = jnp.where(l2 == 0,
                           jnp.broadcast_to(w1v / den, (B, 2)),
                           jnp.broadcast_to(w2v / den, (B, 2))) * ROUTED_SCALING
    oh = ((l16 == e1) | (l16 == e2)).astype(jnp.float32)
    cnt_ref[...] = cnt_ref[...] + jnp.sum(oh, axis=0, keepdims=True)

    @pl.when(b == nsteps - 1)
    def _():
        counts = cnt_ref[...]
        nb = jnp.floor((counts + (BT - 1)) * (1.0 / BT))
        r16 = lax.broadcasted_iota(jnp.int32, (E, E), 0)
        c16 = lax.broadcasted_iota(jnp.int32, (E, E), 1)
        upper = (r16 <= c16).astype(jnp.float32)
        incl = lax.dot_general(nb, upper, (((1,), (0,)), ((), ())),
                               preferred_element_type=jnp.float32)
        bstart = incl - nb
        bases_ref[...] = bstart * BT
        nvalid = jnp.sum(nb, axis=1, keepdims=True)
        eye = (r16 == c16).astype(jnp.float32)
        bstart_col = lax.dot_general(eye, bstart, (((1,), (1,)), ((), ())),
                                     preferred_element_type=jnp.float32)
        iota_b = lax.broadcasted_iota(
            jnp.int32, (E, MAXB_PAD), 1).astype(jnp.float32)
        ind = (iota_b >= bstart_col).astype(jnp.float32)
        ones = jnp.ones((1, E), jnp.float32)
        be = lax.dot_general(ones, ind, (((1,), (0,)), ((), ())),
                             preferred_element_type=jnp.float32) - 1.0
        belast = jnp.sum((bstart < nvalid).astype(jnp.float32),
                         axis=1, keepdims=True) - 1.0
        be = jnp.minimum(be, belast)
        l64 = lax.broadcasted_iota(jnp.int32, (1, MAXB_PAD), 1)
        meta_ref[...] = jnp.where(
            l64 == 48, jnp.broadcast_to(nvalid, (1, MAXB_PAD)),
            be).astype(jnp.int32)


def _gate_call(x, gate_w, gate_bias):
    T, D = x.shape
    E = gate_w.shape[0]
    B1 = 512
    return pl.pallas_call(
        _gate_body,
        grid=(T // B1,),
        in_specs=[
            pl.BlockSpec((B1, D), lambda b: (b, 0)),
            pl.BlockSpec((E, D), lambda b: (0, 0)),
            pl.BlockSpec((1, E), lambda b: (0, 0)),
        ],
        out_specs=[
            pl.BlockSpec((B1, 2), lambda b: (b, 0)),
            pl.BlockSpec((B1, 2), lambda b: (b, 0)),
            pl.BlockSpec((1, E), lambda b: (0, 0)),
            pl.BlockSpec((1, MAXB_PAD), lambda b: (0, 0)),
        ],
        out_shape=[
            jax.ShapeDtypeStruct((T, 2), jnp.int32),
            jax.ShapeDtypeStruct((T, 2), jnp.float32),
            jax.ShapeDtypeStruct((1, E), jnp.float32),
            jax.ShapeDtypeStruct((1, MAXB_PAD), jnp.int32),
        ],
        scratch_shapes=[pltpu.VMEM((1, E), jnp.float32)],
    )(x, gate_w, gate_bias)


def _slot_body(ti_ref, bases_ref, s0_ref, cnt_ref):
    b = pl.program_id(0)
    B = ti_ref.shape[0]
    E = bases_ref.shape[1]

    @pl.when(b == 0)
    def _():
        cnt_ref[...] = jnp.zeros_like(cnt_ref)

    le = lax.broadcasted_iota(jnp.int32, (B, E), 1)
    rr = lax.broadcasted_iota(jnp.int32, (B, B), 0)
    cc = lax.broadcasted_iota(jnp.int32, (B, B), 1)
    ltri = (cc < rr).astype(jnp.float32)
    bases = bases_ref[...]
    oh0 = (ti_ref[:, 0:1] == le).astype(jnp.float32)
    oh1 = (ti_ref[:, 1:2] == le).astype(jnp.float32)
    pref0 = lax.dot_general(ltri, oh0, (((1,), (0,)), ((), ())),
                            preferred_element_type=jnp.float32) + cnt_ref[...]
    s0 = jnp.sum((pref0 + bases) * oh0, axis=1, keepdims=True)
    cnt1 = cnt_ref[...] + jnp.sum(oh0, axis=0, keepdims=True)
    pref1 = lax.dot_general(ltri, oh1, (((1,), (0,)), ((), ())),
                            preferred_element_type=jnp.float32) + cnt1
    s1 = jnp.sum((pref1 + bases) * oh1, axis=1, keepdims=True)
    cnt_ref[...] = cnt1 + jnp.sum(oh1, axis=0, keepdims=True)
    s0_ref[0] = s0.astype(jnp.int32)
    s0_ref[1] = s1.astype(jnp.int32)


def _slot_call(ti, bases):
    T = ti.shape[0]
    E = bases.shape[1]
    B1 = 256
    return pl.pallas_call(
        _slot_body,
        grid=(T // B1,),
        in_specs=[
            pl.BlockSpec((B1, 2), lambda b: (b, 0)),
            pl.BlockSpec((1, E), lambda b: (0, 0)),
        ],
        out_specs=pl.BlockSpec((2, B1, 1), lambda b: (0, b, 0)),
        out_shape=jax.ShapeDtypeStruct((2, T, 1), jnp.int32),
        scratch_shapes=[pltpu.VMEM((1, E), jnp.float32)],
    )(ti, bases)


def _scatter_rows(x, slots, S):
    T, D = x.shape
    t_per_w = T // NW
    mesh = plsc.VectorSubcoreMesh(core_axis_name="c", subcore_axis_name="s")

    @functools.partial(
        pl.kernel, mesh=mesh,
        out_type=jax.ShapeDtypeStruct((S, D), jnp.float32),
        scratch_types=[
            pltpu.VMEM((t_per_w,), jnp.int32),
            pltpu.VMEM((t_per_w,), jnp.int32),
            pltpu.VMEM((t_per_w, D), jnp.float32),
            pltpu.SemaphoreType.DMA,
        ],
    )
    def k(x_hbm, i0_hbm, i1_hbm, out_hbm, i0_v, i1_v, rows_v, sem):
        wid = lax.axis_index("s") * NC + lax.axis_index("c")
        base = wid * t_per_w
        pltpu.sync_copy(x_hbm.at[pl.ds(base, t_per_w)], rows_v)
        pltpu.sync_copy(i0_hbm.at[wid], i0_v)
        pltpu.sync_copy(i1_hbm.at[wid], i1_v)
        pltpu.async_copy(rows_v, out_hbm.at[i0_v], sem)
        pltpu.async_copy(rows_v, out_hbm.at[i1_v], sem)
        pltpu.make_async_copy(rows_v, out_hbm.at[i0_v], sem).wait()
        pltpu.make_async_copy(rows_v, out_hbm.at[i1_v], sem).wait()

    return k(x, slots[0].reshape(NW, t_per_w), slots[1].reshape(NW, t_per_w))


def _gather_rows(table, slots, chunk):
    K, T, _ = slots.shape
    B = K * T
    D = table.shape[1]
    b_per_w = B // NW
    nch = b_per_w // chunk
    mesh = plsc.VectorSubcoreMesh(core_axis_name="c", subcore_axis_name="s")

    @functools.partial(
        pl.kernel, mesh=mesh,
        out_type=jax.ShapeDtypeStruct((B, D), jnp.float32),
        scratch_types=[
            pltpu.VMEM((chunk,), jnp.int32),
            pltpu.VMEM((chunk, D), jnp.float32),
            pltpu.SemaphoreType.DMA,
        ],
    )
    def k(table_hbm, idx_hbm, out_hbm, idx_v, rows_v, sem):
        wid = lax.axis_index("s") * NC + lax.axis_index("c")
        base = wid * b_per_w
        for i in range(nch):
            off = base + i * chunk
            pltpu.sync_copy(idx_hbm.at[pl.ds(off, chunk)], idx_v)
            pltpu.async_copy(table_hbm.at[idx_v], rows_v, sem).wait()
            pltpu.sync_copy(rows_v, out_hbm.at[pl.ds(off, chunk)])

    return k(table, slots.reshape(K * T))


def _expert_body(meta_ref, xs_ref, w1_ref, w2_ref, ys_ref):
    b = pl.program_id(0)
    nvalid = meta_ref[0, 48]

    @pl.when(b < nvalid)
    def _():
        h = lax.dot_general(xs_ref[...], w1_ref[0], (((1,), (1,)), ((), ())),
                            preferred_element_type=jnp.float32)
        h = jnp.maximum(h, 0.0)
        h = h * h
        ys_ref[...] = lax.dot_general(h, w2_ref[0], (((1,), (1,)), ((), ())),
                                      preferred_element_type=jnp.float32)


def _expert_call(meta, xs, w1, w2, maxb):
    S, D = xs.shape
    E, I, _ = w1.shape
    grid_spec = pltpu.PrefetchScalarGridSpec(
        num_scalar_prefetch=1,
        grid=(maxb,),
        in_specs=[
            pl.BlockSpec((BT, D),
                         lambda b, m: (jnp.minimum(b, m[0, 48] - 1), 0)),
            pl.BlockSpec((1, I, D), lambda b, m: (m[0, b], 0, 0)),
            pl.BlockSpec((1, D, I), lambda b, m: (m[0, b], 0, 0)),
        ],
        out_specs=pl.BlockSpec((BT, D),
                               lambda b, m: (jnp.minimum(b, m[0, 48] - 1), 0)),
    )
    return pl.pallas_call(
        _expert_body,
        grid_spec=grid_spec,
        out_shape=jax.ShapeDtypeStruct((S, D), jnp.float32),
    )(meta, xs, w1, w2)


def _shared_body(x_ref, sw1_ref, sw2_ref, sh_ref):
    h = lax.dot_general(x_ref[...], sw1_ref[...], (((1,), (1,)), ((), ())),
                        preferred_element_type=jnp.float32)
    h = jnp.maximum(h, 0.0)
    h = h * h
    sh_ref[...] = lax.dot_general(h, sw2_ref[...], (((1,), (1,)), ((), ())),
                                  preferred_element_type=jnp.float32)


def _shared_call(x, shared_w1, shared_w2):
    T, D = x.shape
    SI = shared_w1.shape[0]
    B1 = 256
    return pl.pallas_call(
        _shared_body,
        grid=(T // B1,),
        in_specs=[
            pl.BlockSpec((B1, D), lambda b: (b, 0)),
            pl.BlockSpec((SI, D), lambda b: (0, 0)),
            pl.BlockSpec((D, SI), lambda b: (0, 0)),
        ],
        out_specs=pl.BlockSpec((B1, D), lambda b: (b, 0)),
        out_shape=jax.ShapeDtypeStruct((T, D), jnp.float32),
    )(x, shared_w1, shared_w2)


def _combine_body(sh_ref, y0_ref, y1_ref, tw_ref, o_ref):
    w = tw_ref[...]
    o_ref[...] = (sh_ref[...] + w[:, 0:1] * y0_ref[...]
                  + w[:, 1:2] * y1_ref[...])


def _combine_call(sh, yg, tw):
    T, D = sh.shape
    nb = T // BT
    return pl.pallas_call(
        _combine_body,
        grid=(nb,),
        in_specs=[
            pl.BlockSpec((BT, D), lambda b: (b, 0)),
            pl.BlockSpec((BT, D), lambda b: (b, 0)),
            pl.BlockSpec((BT, D), lambda b: (b + nb, 0)),
            pl.BlockSpec((BT, 2), lambda b: (b, 0)),
        ],
        out_specs=pl.BlockSpec((BT, D), lambda b: (b, 0)),
        out_shape=jax.ShapeDtypeStruct((T, D), jnp.float32),
    )(sh, yg, yg, tw)


def kernel(hidden_states, gate_w, gate_bias, w1, w2, shared_w1, shared_w2):
    x = hidden_states
    T, D = x.shape
    E = gate_w.shape[0]
    P = 2 * T
    maxb = P // BT + E
    S = maxb * BT

    ti, tw, bases, meta = _gate_call(x, gate_w, gate_bias.reshape(1, E))
    slots = _slot_call(ti, bases)

    xs = _scatter_rows(x, slots, S)
    sh = _shared_call(x, shared_w1, shared_w2)
    ys = _expert_call(meta, xs, w1, w2, maxb)
    yg = _gather_rows(ys, slots, 64)

    return _combine_call(sh, yg, tw)
